# baseline (speedup 1.0000x reference)
import dataclasses
import functools

import numpy as np

import jax
import jax.numpy as jnp
from jax import lax
from jax.experimental import pallas as pl
from jax.experimental.pallas import tpu as pltpu
from jax.experimental.pallas import tpu_sc as plsc

N = 10000
D = 128
NC = 2
NS = 16
NT = NC * NS
CH = 96
NCH = 112
EPAD = NT * NCH * CH
NPAD = 10240
ROWS_PER_TILE = NPAD // NS
CPY = 128


def _sc_segment_sum(x, packed, valb):
    mesh = plsc.VectorSubcoreMesh(core_axis_name="c", subcore_axis_name="s")
    cp = pltpu.CompilerParams()
    if "needs_layout_passes" in pltpu.CompilerParams.__dataclass_fields__:
        cp = dataclasses.replace(cp, needs_layout_passes=False)

    @functools.partial(
        pl.kernel,
        compiler_params=cp,
        out_type=jax.ShapeDtypeStruct((NC, NPAD, D), jnp.float32),
        mesh=mesh,
        scratch_types=[
            pltpu.VMEM((NCH * CH,), jnp.int32),
            pltpu.VMEM((3, CH), jnp.int32),
            pltpu.VMEM((3, CH), jnp.int32),
            pltpu.VMEM((3, 128), jnp.float32),
            pltpu.VMEM((CH, D), jnp.float32),
            pltpu.VMEM((CH, D), jnp.float32),
            pltpu.VMEM((CH, D), jnp.float32),
            pltpu.VMEM_SHARED((NPAD, D), jnp.float32),
            pltpu.SemaphoreType.DMA,
            pltpu.SemaphoreType.DMA,
            pltpu.SemaphoreType.DMA,
            pltpu.SemaphoreType.DMA,
            pltpu.SemaphoreType.DMA,
            pltpu.SemaphoreType.DMA,
        ],
    )
    def sc_kernel(x_hbm, pk_hbm, val_hbm, out_hbm,
                  pk_v, colb, rowb, valc, b0, b1, b2, agg,
                  g0, g1, g2, s0, s1, s2):
        c = lax.axis_index("c")
        s = lax.axis_index("s")
        wid = c * NS + s
        bufs = (b0, b1, b2)
        gsems = (g0, g1, g2)
        ssems = (s0, s1, s2)

        zero = jnp.zeros((16,), jnp.float32)

        @pl.loop(0, CH)
        def _zero_buf(r):
            for k in range(D // 16):
                b0[r, pl.ds(k * 16, 16)] = zero

        for i in range(6):
            base = s * ROWS_PER_TILE + i * CH
            pltpu.sync_copy(b0.at[pl.ds(0, CH)], agg.at[pl.ds(base, CH)])
        pltpu.sync_copy(b0.at[pl.ds(0, 64)],
                        agg.at[pl.ds(s * ROWS_PER_TILE + 6 * CH, 64)])

        pltpu.sync_copy(pk_hbm.at[wid], pk_v)

        def unpack(jj, slot):
            for g in range(CH // 16):
                p = pk_v[pl.ds(jj * CH + g * 16, 16)]
                colb[slot, pl.ds(g * 16, 16)] = p & 0x3FFF
                rowb[slot, pl.ds(g * 16, 16)] = lax.shift_right_logical(p, 14)

        def issue(jj, slot):
            pltpu.async_copy(x_hbm.at[colb.at[slot]], bufs[slot], gsems[slot])
            pltpu.async_copy(val_hbm.at[wid, jj], valc.at[slot], gsems[slot])

        def wait_gather(jj, slot):
            pltpu.make_async_copy(
                x_hbm.at[colb.at[slot]], bufs[slot], gsems[slot]).wait()
            pltpu.make_async_copy(val_hbm.at[wid, jj],
                                  valc.at[slot], gsems[slot]).wait()

        def scale(slot):
            bufp = bufs[slot]
            p16 = jnp.full((16,), slot, jnp.int32)

            @pl.loop(0, CH, step=16)
            def _scale(g):
                for t in range(16):
                    e = g + t
                    v = plsc.load_gather(
                        valc, [p16, jnp.full((16,), e, jnp.int32)])
                    for k in range(D // 16):
                        sl = pl.ds(k * 16, 16)
                        bufp[e, sl] = bufp[e, sl] * v

        def start_scatter(slot):
            pltpu.async_copy(bufs[slot], agg.at[rowb.at[slot]],
                             ssems[slot], add=True)

        def wait_scatter(slot):
            pltpu.make_async_copy(bufs[slot], agg.at[rowb.at[slot]],
                                  ssems[slot]).wait()

        def stage(jj, slot, first=False):
            nslot = (slot + 2) % 3
            if not first:
                wait_scatter(nslot)
            unpack(jj + 2, nslot)
            issue(jj + 2, nslot)
            wait_gather(jj, slot)
            scale(slot)
            start_scatter(slot)

        unpack(0, 0)
        issue(0, 0)
        unpack(1, 1)
        issue(1, 1)
        stage(0, 0, first=True)
        stage(1, 1)

        @pl.loop(2, NCH - 2, step=3)
        def _chunk(j):
            stage(j, 2)
            stage(j + 1, 0)
            stage(j + 2, 1)

        wait_gather(NCH - 2, 2)
        scale(2)
        start_scatter(2)
        wait_gather(NCH - 1, 0)
        scale(0)
        start_scatter(0)
        wait_scatter(1)
        wait_scatter(2)
        wait_scatter(0)

        plsc.subcore_barrier()
        for i in range(ROWS_PER_TILE // CPY):
            st = s * ROWS_PER_TILE + i * CPY
            pltpu.sync_copy(agg.at[pl.ds(st, CPY)],
                            out_hbm.at[c, pl.ds(st, CPY)])

    return sc_kernel(x, packed, valb)


def _mm_body(p0_ref, p1_ref, w_ref, b_ref, o_ref):
    acc = p0_ref[...] + p1_ref[...]
    o_ref[...] = lax.dot(acc, w_ref[...],
                         preferred_element_type=jnp.float32) + b_ref[...]


def _tc_matmul(p0, p1, W, b2):
    blk = 1000
    return pl.pallas_call(
        _mm_body,
        grid=(N // blk,),
        in_specs=[
            pl.BlockSpec((blk, D), lambda i: (i, 0)),
            pl.BlockSpec((blk, D), lambda i: (i, 0)),
            pl.BlockSpec((D, D), lambda i: (0, 0)),
            pl.BlockSpec((1, D), lambda i: (0, 0)),
        ],
        out_specs=pl.BlockSpec((blk, D), lambda i: (i, 0)),
        out_shape=jax.ShapeDtypeStruct((N, D), jnp.float32),
    )(p0, p1, W, b2)


def kernel(x, edge_index, edge_vals, W, b):
    row = edge_index[0]
    col = edge_index[1]
    e = row.shape[0]
    pad = EPAD - e
    spread_np = np.arange(pad, dtype=np.int32) % N
    pad_packed = jnp.asarray((spread_np << 14) | spread_np)
    packed = jnp.concatenate([jnp.left_shift(row, 14) | col, pad_packed])
    packed = packed.reshape(NT, NCH * CH)
    valp = jnp.concatenate([edge_vals, jnp.zeros((pad,), jnp.float32)])
    val3 = jnp.pad(valp.reshape(NT, NCH, CH), ((0, 0), (0, 0), (0, 128 - CH)))

    partials = _sc_segment_sum(x, packed, val3)
    return _tc_matmul(partials[0], partials[1], W, b.reshape(1, D))


# speedup vs baseline: 1.3089x; 1.3089x over previous
import dataclasses
import functools

import numpy as np

import jax
import jax.numpy as jnp
from jax import lax
from jax.experimental import pallas as pl
from jax.experimental.pallas import tpu as pltpu
from jax.experimental.pallas import tpu_sc as plsc

N = 10000
D = 128
NC = 2
NS = 16
NT = NC * NS
CH = 64
NCH = 160
EPAD = NT * NCH * CH
NPAD = 10240
ROWS_PER_TILE = NPAD // NS
CPY = 128


def _sc_segment_sum(x, packed, valb):
    mesh = plsc.VectorSubcoreMesh(core_axis_name="c", subcore_axis_name="s")
    cp = pltpu.CompilerParams()
    if "needs_layout_passes" in pltpu.CompilerParams.__dataclass_fields__:
        cp = dataclasses.replace(cp, needs_layout_passes=False)

    @functools.partial(
        pl.kernel,
        compiler_params=cp,
        out_type=jax.ShapeDtypeStruct((NC, NPAD, D), jnp.float32),
        mesh=mesh,
        scratch_types=[
            pltpu.VMEM((NCH * CH,), jnp.int32),
            pltpu.VMEM((4, CH), jnp.int32),
            pltpu.VMEM((4, CH), jnp.int32),
            pltpu.VMEM((4, CH), jnp.float32),
            pltpu.VMEM((CH, D), jnp.float32),
            pltpu.VMEM((CH, D), jnp.float32),
            pltpu.VMEM((CH, D), jnp.float32),
            pltpu.VMEM((CH, D), jnp.float32),
            pltpu.VMEM_SHARED((NPAD, D), jnp.float32),
            pltpu.SemaphoreType.DMA,
            pltpu.SemaphoreType.DMA,
            pltpu.SemaphoreType.DMA,
            pltpu.SemaphoreType.DMA,
            pltpu.SemaphoreType.DMA,
            pltpu.SemaphoreType.DMA,
            pltpu.SemaphoreType.DMA,
            pltpu.SemaphoreType.DMA,
        ],
    )
    def sc_kernel(x_hbm, pk_hbm, val_hbm, out_hbm,
                  pk_v, colb, rowb, valc, b0, b1, b2, b3, agg,
                  g0, g1, g2, g3, s0, s1, s2, s3):
        c = lax.axis_index("c")
        s = lax.axis_index("s")
        wid = c * NS + s
        bufs = (b0, b1, b2, b3)
        gsems = (g0, g1, g2, g3)
        ssems = (s0, s1, s2, s3)

        zero = jnp.zeros((16,), jnp.float32)

        @pl.loop(0, CH)
        def _zero_buf(r):
            for k in range(D // 16):
                b0[r, pl.ds(k * 16, 16)] = zero

        for i in range(ROWS_PER_TILE // CPY):
            base = s * ROWS_PER_TILE + i * CPY
            pltpu.sync_copy(b0.at[pl.ds(0, CH)], agg.at[pl.ds(base, CH)])
            pltpu.sync_copy(b0.at[pl.ds(0, CH)], agg.at[pl.ds(base + CH, CH)])

        pltpu.sync_copy(pk_hbm.at[wid], pk_v)

        def unpack(jj, slot):
            for g in range(CH // 16):
                p = pk_v[pl.ds(jj * CH + g * 16, 16)]
                colb[slot, pl.ds(g * 16, 16)] = p & 0x3FFF
                rowb[slot, pl.ds(g * 16, 16)] = lax.shift_right_logical(p, 14)

        def issue(jj, slot):
            pltpu.async_copy(x_hbm.at[colb.at[slot]], bufs[slot], gsems[slot])
            pltpu.async_copy(val_hbm.at[wid, pl.ds(jj * CH, CH)],
                             valc.at[slot], gsems[slot])

        def wait_gather(jj, slot):
            pltpu.make_async_copy(
                x_hbm.at[colb.at[slot]], bufs[slot], gsems[slot]).wait()
            pltpu.make_async_copy(val_hbm.at[wid, pl.ds(jj * CH, CH)],
                                  valc.at[slot], gsems[slot]).wait()

        def scale(slot):
            bufp = bufs[slot]
            p16 = jnp.full((16,), slot, jnp.int32)

            @pl.loop(0, CH, step=16)
            def _scale(g):
                for t in range(16):
                    e = g + t
                    v = plsc.load_gather(
                        valc, [p16, jnp.full((16,), e, jnp.int32)])
                    for k in range(D // 16):
                        sl = pl.ds(k * 16, 16)
                        bufp[e, sl] = bufp[e, sl] * v

        def start_scatter(slot):
            pltpu.async_copy(bufs[slot], agg.at[rowb.at[slot]],
                             ssems[slot], add=True)

        def wait_scatter(slot):
            pltpu.make_async_copy(bufs[slot], agg.at[rowb.at[slot]],
                                  ssems[slot]).wait()

        def stage(jj, slot, first):
            nslot = (slot + 2) % 4
            if not first:
                wait_scatter(nslot)
            unpack(jj + 2, nslot)
            issue(jj + 2, nslot)
            wait_gather(jj, slot)
            scale(slot)
            start_scatter(slot)

        unpack(0, 0)
        issue(0, 0)
        unpack(1, 1)
        issue(1, 1)
        stage(0, 0, True)
        stage(1, 1, True)

        @pl.loop(2, NCH - 2, step=4)
        def _chunk(j):
            stage(j, 2, False)
            stage(j + 1, 3, False)
            stage(j + 2, 0, False)
            stage(j + 3, 1, False)

        wait_scatter(0)
        wait_gather(NCH - 2, 2)
        scale(2)
        start_scatter(2)
        wait_scatter(1)
        wait_gather(NCH - 1, 3)
        scale(3)
        start_scatter(3)
        wait_scatter(2)
        wait_scatter(3)

        plsc.subcore_barrier()
        for i in range(ROWS_PER_TILE // CPY):
            st = s * ROWS_PER_TILE + i * CPY
            pltpu.sync_copy(agg.at[pl.ds(st, CPY)],
                            out_hbm.at[c, pl.ds(st, CPY)])

    return sc_kernel(x, packed, valb)


def _mm_body(p0_ref, p1_ref, w_ref, b_ref, o_ref):
    acc = p0_ref[...] + p1_ref[...]
    o_ref[...] = lax.dot(acc, w_ref[...],
                         preferred_element_type=jnp.float32) + b_ref[...]


def _tc_matmul(p0, p1, W, b2):
    blk = 1000
    return pl.pallas_call(
        _mm_body,
        grid=(N // blk,),
        in_specs=[
            pl.BlockSpec((blk, D), lambda i: (i, 0)),
            pl.BlockSpec((blk, D), lambda i: (i, 0)),
            pl.BlockSpec((D, D), lambda i: (0, 0)),
            pl.BlockSpec((1, D), lambda i: (0, 0)),
        ],
        out_specs=pl.BlockSpec((blk, D), lambda i: (i, 0)),
        out_shape=jax.ShapeDtypeStruct((N, D), jnp.float32),
    )(p0, p1, W, b2)


def kernel(x, edge_index, edge_vals, W, b):
    row = edge_index[0]
    col = edge_index[1]
    e = row.shape[0]
    pad = EPAD - e
    spread_np = np.arange(pad, dtype=np.int32) % N
    pad_packed = jnp.asarray((spread_np << 14) | spread_np)
    packed = jnp.concatenate([jnp.left_shift(row, 14) | col, pad_packed])
    packed = packed.reshape(NT, NCH * CH)
    valp = jnp.concatenate([edge_vals, jnp.zeros((pad,), jnp.float32)])
    val2 = valp.reshape(NT, NCH * CH)

    partials = _sc_segment_sum(x, packed, val2)
    return _tc_matmul(partials[0], partials[1], W, b.reshape(1, D))
